# baseline (speedup 1.0000x reference)
import functools

import jax
import jax.numpy as jnp
from jax import lax
from jax.experimental import pallas as pl
from jax.experimental.pallas import tpu as pltpu
from jax.experimental.pallas import tpu_sc as plsc

NC, NS, L = 2, 16, 16
NW = NC * NS
NUM_IDX = 16384
D = 128
PER_W = NUM_IDX // NW
SIZES = (32, 96, 128, 128, 128)
OFFS = (0, 32, 128, 256, 384)
G = D // L

_mesh = plsc.VectorSubcoreMesh(
    core_axis_name="c", subcore_axis_name="s", num_cores=NC, num_subcores=NS)


@functools.partial(
    pl.kernel,
    out_type=jax.ShapeDtypeStruct((NC, L), jnp.float32),
    mesh=_mesh,
    compiler_params=pltpu.CompilerParams(
        needs_layout_passes=False,
        disable_bounds_checks=True,
        disable_semaphore_checks=True,
        skip_device_barrier=True,
    ),
    scratch_types=[
        pltpu.VMEM((PER_W,), jnp.int32),
        pltpu.VMEM((PER_W, D), jnp.float32),
        pltpu.VMEM((NS, D), jnp.float32),
        pltpu.VMEM((D,), jnp.int32),
        pltpu.VMEM((D,), jnp.float32),
        pltpu.VMEM((L,), jnp.float32),
        pltpu.VMEM_SHARED((NS, D), jnp.float32),
        pltpu.SemaphoreType.DMA,
        pltpu.SemaphoreType.DMA,
        pltpu.SemaphoreType.DMA,
    ],
)
def _sum_kernel(a_hbm, b_hbm, coef_hbm, out_hbm,
                idx_v, bufs, gath_v, bidx_v, colsum_v, out_v,
                shared, isem, sem, bsem):
    cid = lax.axis_index("c")
    sid = lax.axis_index("s")
    wid = cid * NS + sid
    base = wid * PER_W

    @pl.when(sid == 0)
    def _():
        pltpu.async_copy(b_hbm, bidx_v, bsem)

    ihandles = [
        pltpu.async_copy(a_hbm.at[pl.ds(base + o, n)],
                         idx_v.at[pl.ds(o, n)], isem)
        for o, n in zip(OFFS, SIZES)
    ]
    handles = []
    for i, (o, n) in enumerate(zip(OFFS, SIZES)):
        ihandles[i].wait()
        handles.append(
            pltpu.async_copy(coef_hbm.at[idx_v.at[pl.ds(o, n)]],
                             bufs.at[pl.ds(o, n)], sem))

    ROW_U = 8

    def accum(o, n, accs):
        def body(r0, a):
            a = list(a)
            for u in range(ROW_U):
                r = o + r0 * ROW_U + u
                for g in range(G):
                    a[g] = a[g] + bufs[r, pl.ds(g * L, L)]
            return tuple(a)
        return lax.fori_loop(0, n // ROW_U, body, accs)

    accs = tuple(jnp.zeros((L,), jnp.float32) for _ in range(G))
    for i, (o, n) in enumerate(zip(OFFS, SIZES)):
        handles[i].wait()
        accs = accum(o, n, accs)

    for g in range(G):
        colsum_v[pl.ds(g * L, L)] = accs[g]
    pltpu.sync_copy(colsum_v, shared.at[sid])
    plsc.subcore_barrier()

    @pl.when(sid == 0)
    def _():
        pltpu.sync_copy(shared, gath_v)
        pltpu.make_async_copy(b_hbm, bidx_v, bsem).wait()
        for g in range(G):
            acc = gath_v[0, pl.ds(g * L, L)]
            for s in range(1, NS):
                acc = acc + gath_v[s, pl.ds(g * L, L)]
            colsum_v[pl.ds(g * L, L)] = acc
        tot = jnp.zeros((L,), jnp.float32)
        for g in range(G):
            idxg = bidx_v[pl.ds(g * L, L)]
            tot = tot + plsc.load_gather(colsum_v, [idxg])
        s_val = jnp.sum(tot)
        lane = lax.iota(jnp.int32, L)
        out_v[...] = jnp.where(lane == 0, s_val, jnp.float32(0.0))
        pltpu.sync_copy(out_v, out_hbm.at[cid])


def kernel(a_indices, b_indices, coefficients):
    out = _sum_kernel(a_indices.astype(jnp.int32),
                      b_indices.astype(jnp.int32),
                      coefficients)
    return jnp.sum(out)


# speedup vs baseline: 1.0345x; 1.0345x over previous
import functools

import jax
import jax.numpy as jnp
from jax import lax
from jax.experimental import pallas as pl
from jax.experimental.pallas import tpu as pltpu
from jax.experimental.pallas import tpu_sc as plsc

NC, NS, L = 2, 16, 16
NW = NC * NS
NUM_IDX = 16384
D = 128
PER_W = NUM_IDX // NW
SIZES = (32, 96, 128, 128, 128)
OFFS = (0, 32, 128, 256, 384)
G = D // L

_mesh = plsc.VectorSubcoreMesh(
    core_axis_name="c", subcore_axis_name="s", num_cores=NC, num_subcores=NS)


@functools.partial(
    pl.kernel,
    out_type=jax.ShapeDtypeStruct((NC, L), jnp.float32),
    mesh=_mesh,
    compiler_params=pltpu.CompilerParams(
        needs_layout_passes=False,
        disable_bounds_checks=True,
        disable_semaphore_checks=True,
        skip_device_barrier=True,
    ),
    scratch_types=[
        pltpu.VMEM((PER_W,), jnp.int32),
        pltpu.VMEM((PER_W, D), jnp.float32),
        pltpu.VMEM((NS, D), jnp.float32),
        pltpu.VMEM((D,), jnp.int32),
        pltpu.VMEM((D,), jnp.float32),
        pltpu.VMEM((L,), jnp.float32),
        pltpu.VMEM_SHARED((NS, D), jnp.float32),
        pltpu.SemaphoreType.DMA,
        pltpu.SemaphoreType.DMA,
        pltpu.SemaphoreType.DMA,
    ],
)
def _sum_kernel(a_hbm, b_hbm, coef_hbm, out_hbm,
                idx_v, bufs, gath_v, bidx_v, colsum_v, out_v,
                shared, isem, sem, bsem):
    cid = lax.axis_index("c")
    sid = lax.axis_index("s")
    wid = cid * NS + sid
    base = wid * PER_W

    @pl.when(sid == 0)
    def _():
        pltpu.async_copy(b_hbm, bidx_v, bsem)

    ihandles = [
        pltpu.async_copy(a_hbm.at[pl.ds(base + o, n)],
                         idx_v.at[pl.ds(o, n)], isem)
        for o, n in zip(OFFS, SIZES)
    ]
    handles = []
    for i, (o, n) in enumerate(zip(OFFS, SIZES)):
        ihandles[i].wait()
        handles.append(
            pltpu.async_copy(coef_hbm.at[idx_v.at[pl.ds(o, n)]],
                             bufs.at[pl.ds(o, n)], sem))

    ROW_U = 4

    def accum(o, n, accs):
        def body(r0, a):
            a = list(a)
            for u in range(ROW_U):
                r = o + r0 * ROW_U + u
                for g in range(G):
                    a[g] = a[g] + bufs[r, pl.ds(g * L, L)]
            return tuple(a)
        return lax.fori_loop(0, n // ROW_U, body, accs)

    accs = tuple(jnp.zeros((L,), jnp.float32) for _ in range(G))
    for i, (o, n) in enumerate(zip(OFFS, SIZES)):
        handles[i].wait()
        accs = accum(o, n, accs)

    for g in range(G):
        colsum_v[pl.ds(g * L, L)] = accs[g]
    pltpu.sync_copy(colsum_v, shared.at[sid])
    plsc.subcore_barrier()

    @pl.when(sid == 0)
    def _():
        pltpu.sync_copy(shared, gath_v)
        pltpu.make_async_copy(b_hbm, bidx_v, bsem).wait()
        for g in range(G):
            acc = gath_v[0, pl.ds(g * L, L)]
            for s in range(1, NS):
                acc = acc + gath_v[s, pl.ds(g * L, L)]
            colsum_v[pl.ds(g * L, L)] = acc
        tot = jnp.zeros((L,), jnp.float32)
        for g in range(G):
            idxg = bidx_v[pl.ds(g * L, L)]
            tot = tot + plsc.load_gather(colsum_v, [idxg])
        s_val = jnp.sum(tot)
        lane = lax.iota(jnp.int32, L)
        out_v[...] = jnp.where(lane == 0, s_val, jnp.float32(0.0))
        pltpu.sync_copy(out_v, out_hbm.at[cid])


def kernel(a_indices, b_indices, coefficients):
    out = _sum_kernel(a_indices.astype(jnp.int32),
                      b_indices.astype(jnp.int32),
                      coefficients)
    return jnp.sum(out)
